# Initial kernel scaffold; baseline (speedup 1.0000x reference)
import jax
import jax.numpy as jnp
from jax.experimental import pallas as pl


def kernel(x, edge_index, W1, b1, W2, b2, Wa, ba, Wf, bf):
    raise NotImplementedError("write your pallas kernel here")


# speedup vs baseline: 20.6545x; 20.6545x over previous
import functools

import jax
import jax.numpy as jnp
from jax import lax
from jax.experimental import pallas as pl
from jax.experimental.pallas import tpu as pltpu
from jax.experimental.pallas import tpu_sc as plsc

N_NODES = 10000
N_EDGES = 320000
D = 128
LABEL_DIM = 64

NW = 32
CHUNK = 128
KCH = 79
EPW = N_EDGES // NW
PAD_PW = KCH * CHUNK - EPW
N_PAD = 10240
RPT = N_PAD // 16

_sc_mesh = plsc.VectorSubcoreMesh(core_axis_name="c", subcore_axis_name="s")


def _deg_body(dst_hbm, ones_hbm, zvec_hbm, out_hbm, dst_v, ones_v, deg_sh):
    cid = lax.axis_index("c")
    sid = lax.axis_index("s")
    wid = sid * 2 + cid
    pltpu.sync_copy(zvec_hbm, deg_sh.at[pl.ds(sid * RPT, RPT)])
    pltpu.sync_copy(ones_hbm, ones_v)
    pltpu.sync_copy(dst_hbm.at[wid], dst_v)
    plsc.subcore_barrier()

    def body(j, carry):
        pltpu.sync_copy(ones_v, deg_sh.at[dst_v.at[j]], add=True)
        return carry

    lax.fori_loop(0, KCH, body, 0)
    plsc.subcore_barrier()
    pltpu.sync_copy(deg_sh.at[pl.ds(sid * RPT, RPT)],
                    out_hbm.at[pl.ds(cid * N_PAD + sid * RPT, RPT)])


_deg_call = functools.partial(
    pl.kernel,
    out_type=jax.ShapeDtypeStruct((2 * N_PAD,), jnp.float32),
    mesh=_sc_mesh,
    scratch_types=[
        pltpu.VMEM((KCH, CHUNK), jnp.int32),
        pltpu.VMEM((CHUNK,), jnp.float32),
        pltpu.VMEM_SHARED((N_PAD,), jnp.float32),
    ],
)(_deg_body)


def _scat_body(g_hbm, src_hbm, dst_hbm, zrows_hbm, out_hbm,
               src_v, dst_v, rows_v, acc_sh, sem):
    cid = lax.axis_index("c")
    sid = lax.axis_index("s")
    wid = sid * 2 + cid
    pltpu.sync_copy(zrows_hbm, acc_sh.at[pl.ds(sid * RPT, RPT)])
    pltpu.sync_copy(src_hbm.at[wid], src_v)
    pltpu.sync_copy(dst_hbm.at[wid], dst_v)
    plsc.subcore_barrier()

    def body(j, carry):
        pltpu.async_copy(g_hbm.at[src_v.at[j]], rows_v, sem).wait()
        pltpu.sync_copy(rows_v, acc_sh.at[dst_v.at[j]], add=True)
        return carry

    lax.fori_loop(0, KCH, body, 0)
    plsc.subcore_barrier()
    pltpu.sync_copy(acc_sh.at[pl.ds(sid * RPT, RPT)],
                    out_hbm.at[pl.ds(cid * N_PAD + sid * RPT, RPT)])


_scat_call = functools.partial(
    pl.kernel,
    out_type=jax.ShapeDtypeStruct((2 * N_PAD, D), jnp.float32),
    mesh=_sc_mesh,
    scratch_types=[
        pltpu.VMEM((KCH, CHUNK), jnp.int32),
        pltpu.VMEM((KCH, CHUNK), jnp.int32),
        pltpu.VMEM((CHUNK, D), jnp.float32),
        pltpu.VMEM_SHARED((N_PAD, D), jnp.float32),
        pltpu.SemaphoreType.DMA,
    ],
)(_scat_body)


def _tc_a_body(x_ref, w1_ref, dega_ref, degb_ref, g_ref, dis_ref):
    deg = dega_ref[...] + degb_ref[...] + 1.0
    dis = lax.rsqrt(deg)
    h = jnp.dot(x_ref[...], w1_ref[...], preferred_element_type=jnp.float32)
    g_ref[...] = h * dis
    dis_ref[...] = dis


_tc_a = pl.pallas_call(
    _tc_a_body,
    out_shape=(jax.ShapeDtypeStruct((N_PAD, D), jnp.float32),
               jax.ShapeDtypeStruct((N_PAD, 1), jnp.float32)),
)


def _tc_c_body(acca_ref, accb_ref, g1_ref, dis_ref, b1_ref, w2_ref, g2_ref):
    dis = dis_ref[...]
    h1 = (acca_ref[...] + accb_ref[...] + g1_ref[...]) * dis + b1_ref[...]
    h1 = jnp.maximum(h1, 0.0)
    g2_ref[...] = jnp.dot(h1, w2_ref[...], preferred_element_type=jnp.float32) * dis


_tc_c = pl.pallas_call(
    _tc_c_body,
    out_shape=jax.ShapeDtypeStruct((N_PAD, D), jnp.float32),
)


def _tc_e_body(acca_ref, accb_ref, g2_ref, dis_ref, b2_ref,
               wa_ref, ba_ref, wf_ref, bf_ref, out_ref):
    dis = dis_ref[...]
    h2 = (acca_ref[...] + accb_ref[...] + g2_ref[...]) * dis + b2_ref[...]
    h2 = jnp.maximum(h2, 0.0)
    logits = jnp.dot(h2, wa_ref[...], preferred_element_type=jnp.float32) + ba_ref[...]
    row = lax.broadcasted_iota(jnp.int32, (N_PAD, 1), 0)
    logits = jnp.where(row < N_NODES, logits, -1e30)
    m = jnp.max(logits)
    w = jnp.exp(logits - m)
    s = jnp.sum(w)
    pooled = jnp.sum(w * h2, axis=0, keepdims=True) / s
    out_ref[...] = jnp.dot(pooled, wf_ref[...],
                           preferred_element_type=jnp.float32) + bf_ref[...]


_tc_e = pl.pallas_call(
    _tc_e_body,
    out_shape=jax.ShapeDtypeStruct((1, LABEL_DIM), jnp.float32),
)


def _pad_edges(v):
    v2 = v.reshape(NW, EPW)
    padc = jnp.broadcast_to(
        (N_NODES + jnp.arange(NW, dtype=jnp.int32))[:, None], (NW, PAD_PW))
    return jnp.concatenate([v2, padc], axis=1).reshape(NW, KCH, CHUNK)


def kernel(x, edge_index, W1, b1, W2, b2, Wa, ba, Wf, bf):
    src_p = _pad_edges(edge_index[0].astype(jnp.int32))
    dst_p = _pad_edges(edge_index[1].astype(jnp.int32))
    ones = jnp.ones((CHUNK,), jnp.float32)
    zvec = jnp.zeros((RPT,), jnp.float32)
    zrows = jnp.zeros((RPT, D), jnp.float32)

    deg2 = _deg_call(dst_p, ones, zvec)
    dega = deg2[:N_PAD].reshape(N_PAD, 1)
    degb = deg2[N_PAD:].reshape(N_PAD, 1)

    xp = jnp.pad(x, ((0, N_PAD - N_NODES), (0, 0)))
    g1, dis = _tc_a(xp, W1, dega, degb)

    acc1 = _scat_call(g1, src_p, dst_p, zrows)
    g2 = _tc_c(acc1[:N_PAD], acc1[N_PAD:], g1, dis, b1.reshape(1, D), W2)

    acc2 = _scat_call(g2, src_p, dst_p, zrows)
    out = _tc_e(acc2[:N_PAD], acc2[N_PAD:], g2, dis, b2.reshape(1, D),
                Wa, ba.reshape(1, 1), Wf, bf.reshape(1, LABEL_DIM))
    return out.reshape(LABEL_DIM)


# speedup vs baseline: 26.8050x; 1.2978x over previous
import functools

import jax
import jax.numpy as jnp
from jax import lax
from jax.experimental import pallas as pl
from jax.experimental.pallas import tpu as pltpu
from jax.experimental.pallas import tpu_sc as plsc

N_NODES = 10000
N_EDGES = 320000
D = 128
LABEL_DIM = 64

NW = 32
CHUNK = 128
KCH = 80
HKCH = 40
EPW = N_EDGES // NW
PAD_PW = KCH * CHUNK - EPW
N_PAD = 10240
RPT = N_PAD // 16

_sc_mesh = plsc.VectorSubcoreMesh(core_axis_name="c", subcore_axis_name="s")


def _deg_body(dst_hbm, ones_hbm, zvec_hbm, out_hbm, dst_v, ones_v, deg_sh):
    cid = lax.axis_index("c")
    sid = lax.axis_index("s")
    wid = sid * 2 + cid
    pltpu.sync_copy(zvec_hbm, deg_sh.at[pl.ds(sid * RPT, RPT)])
    pltpu.sync_copy(ones_hbm, ones_v)
    pltpu.sync_copy(dst_hbm.at[wid], dst_v)
    plsc.subcore_barrier()

    def body(j, carry):
        pltpu.sync_copy(ones_v, deg_sh.at[dst_v.at[j]], add=True)
        return carry

    lax.fori_loop(0, KCH, body, 0)
    plsc.subcore_barrier()
    pltpu.sync_copy(deg_sh.at[pl.ds(sid * RPT, RPT)],
                    out_hbm.at[pl.ds(cid * N_PAD + sid * RPT, RPT)])


_deg_call = functools.partial(
    pl.kernel,
    out_type=jax.ShapeDtypeStruct((2 * N_PAD,), jnp.float32),
    mesh=_sc_mesh,
    scratch_types=[
        pltpu.VMEM((KCH, CHUNK), jnp.int32),
        pltpu.VMEM((CHUNK,), jnp.float32),
        pltpu.VMEM_SHARED((N_PAD,), jnp.float32),
    ],
)(_deg_body)


def _scat_body(g_hbm, src_hbm, dst_hbm, zrows_hbm, out_hbm,
               src_v, dst_v, rows0_v, rows1_v, acc_sh, sem0, sem1):
    cid = lax.axis_index("c")
    sid = lax.axis_index("s")
    wid = sid * 2 + cid
    pltpu.sync_copy(zrows_hbm, acc_sh.at[pl.ds(sid * RPT, RPT)])
    plsc.subcore_barrier()

    for p in range(KCH // HKCH):
        pltpu.sync_copy(src_hbm.at[wid, pl.ds(p * HKCH, HKCH)], src_v)
        pltpu.sync_copy(dst_hbm.at[wid, pl.ds(p * HKCH, HKCH)], dst_v)
        pltpu.async_copy(g_hbm.at[src_v.at[0]], rows0_v, sem0)

        def body(jj, carry):
            j0 = 2 * jj
            pltpu.async_copy(g_hbm.at[src_v.at[j0 + 1]], rows1_v, sem1)
            pltpu.make_async_copy(g_hbm.at[src_v.at[j0]], rows0_v, sem0).wait()
            pltpu.sync_copy(rows0_v, acc_sh.at[dst_v.at[j0]], add=True)

            @pl.when(jj < HKCH // 2 - 1)
            def _():
                pltpu.async_copy(g_hbm.at[src_v.at[j0 + 2]], rows0_v, sem0)

            pltpu.make_async_copy(g_hbm.at[src_v.at[j0 + 1]], rows1_v, sem1).wait()
            pltpu.sync_copy(rows1_v, acc_sh.at[dst_v.at[j0 + 1]], add=True)
            return carry

        lax.fori_loop(0, HKCH // 2, body, 0)
    plsc.subcore_barrier()
    pltpu.sync_copy(acc_sh.at[pl.ds(sid * RPT, RPT)],
                    out_hbm.at[pl.ds(cid * N_PAD + sid * RPT, RPT)])


_scat_call = functools.partial(
    pl.kernel,
    out_type=jax.ShapeDtypeStruct((2 * N_PAD, D), jnp.float32),
    mesh=_sc_mesh,
    scratch_types=[
        pltpu.VMEM((HKCH, CHUNK), jnp.int32),
        pltpu.VMEM((HKCH, CHUNK), jnp.int32),
        pltpu.VMEM((CHUNK, D), jnp.float32),
        pltpu.VMEM((CHUNK, D), jnp.float32),
        pltpu.VMEM_SHARED((N_PAD, D), jnp.float32),
        pltpu.SemaphoreType.DMA,
        pltpu.SemaphoreType.DMA,
    ],
)(_scat_body)


def _tc_a_body(x_ref, w1_ref, dega_ref, degb_ref, g_ref, dis_ref):
    deg = dega_ref[...] + degb_ref[...] + 1.0
    dis = lax.rsqrt(deg)
    h = jnp.dot(x_ref[...], w1_ref[...], preferred_element_type=jnp.float32)
    g_ref[...] = h * dis
    dis_ref[...] = dis


_tc_a = pl.pallas_call(
    _tc_a_body,
    out_shape=(jax.ShapeDtypeStruct((N_PAD, D), jnp.float32),
               jax.ShapeDtypeStruct((N_PAD, 1), jnp.float32)),
)


def _tc_c_body(acca_ref, accb_ref, g1_ref, dis_ref, b1_ref, w2_ref, g2_ref):
    dis = dis_ref[...]
    h1 = (acca_ref[...] + accb_ref[...] + g1_ref[...]) * dis + b1_ref[...]
    h1 = jnp.maximum(h1, 0.0)
    g2_ref[...] = jnp.dot(h1, w2_ref[...], preferred_element_type=jnp.float32) * dis


_tc_c = pl.pallas_call(
    _tc_c_body,
    out_shape=jax.ShapeDtypeStruct((N_PAD, D), jnp.float32),
)


def _tc_e_body(acca_ref, accb_ref, g2_ref, dis_ref, b2_ref,
               wa_ref, ba_ref, wf_ref, bf_ref, out_ref):
    dis = dis_ref[...]
    h2 = (acca_ref[...] + accb_ref[...] + g2_ref[...]) * dis + b2_ref[...]
    h2 = jnp.maximum(h2, 0.0)
    logits = jnp.dot(h2, wa_ref[...], preferred_element_type=jnp.float32) + ba_ref[...]
    row = lax.broadcasted_iota(jnp.int32, (N_PAD, 1), 0)
    logits = jnp.where(row < N_NODES, logits, -1e30)
    m = jnp.max(logits)
    w = jnp.exp(logits - m)
    s = jnp.sum(w)
    pooled = jnp.sum(w * h2, axis=0, keepdims=True) / s
    out_ref[...] = jnp.dot(pooled, wf_ref[...],
                           preferred_element_type=jnp.float32) + bf_ref[...]


_tc_e = pl.pallas_call(
    _tc_e_body,
    out_shape=jax.ShapeDtypeStruct((1, LABEL_DIM), jnp.float32),
)


def _pad_edges(v):
    v2 = v.reshape(NW, EPW)
    padc = jnp.broadcast_to(
        (N_NODES + jnp.arange(NW, dtype=jnp.int32))[:, None], (NW, PAD_PW))
    return jnp.concatenate([v2, padc], axis=1).reshape(NW, KCH, CHUNK)


def kernel(x, edge_index, W1, b1, W2, b2, Wa, ba, Wf, bf):
    src_p = _pad_edges(edge_index[0].astype(jnp.int32))
    dst_p = _pad_edges(edge_index[1].astype(jnp.int32))
    ones = jnp.ones((CHUNK,), jnp.float32)
    zvec = jnp.zeros((RPT,), jnp.float32)
    zrows = jnp.zeros((RPT, D), jnp.float32)

    deg2 = _deg_call(dst_p, ones, zvec)
    dega = deg2[:N_PAD].reshape(N_PAD, 1)
    degb = deg2[N_PAD:].reshape(N_PAD, 1)

    xp = jnp.pad(x, ((0, N_PAD - N_NODES), (0, 0)))
    g1, dis = _tc_a(xp, W1, dega, degb)

    acc1 = _scat_call(g1, src_p, dst_p, zrows)
    g2 = _tc_c(acc1[:N_PAD], acc1[N_PAD:], g1, dis, b1.reshape(1, D), W2)

    acc2 = _scat_call(g2, src_p, dst_p, zrows)
    out = _tc_e(acc2[:N_PAD], acc2[N_PAD:], g2, dis, b2.reshape(1, D),
                Wa, ba.reshape(1, 1), Wf, bf.reshape(1, LABEL_DIM))
    return out.reshape(LABEL_DIM)


# speedup vs baseline: 26.8934x; 1.0033x over previous
import functools

import jax
import jax.numpy as jnp
from jax import lax
from jax.experimental import pallas as pl
from jax.experimental.pallas import tpu as pltpu
from jax.experimental.pallas import tpu_sc as plsc

N_NODES = 10000
N_EDGES = 320000
D = 128
LABEL_DIM = 64

NW = 32
CHUNK = 128
KCH = 80
HKCH = 40
EPW = N_EDGES // NW
PAD_PW = KCH * CHUNK - EPW
N_PAD = 10240
RPT = N_PAD // 16

_sc_mesh = plsc.VectorSubcoreMesh(core_axis_name="c", subcore_axis_name="s")


def _deg_body(dst_hbm, ones_hbm, zvec_hbm, out_hbm, dst_v, ones_v, deg_sh):
    cid = lax.axis_index("c")
    sid = lax.axis_index("s")
    wid = sid * 2 + cid
    pltpu.sync_copy(zvec_hbm, deg_sh.at[pl.ds(sid * RPT, RPT)])
    pltpu.sync_copy(ones_hbm, ones_v)
    pltpu.sync_copy(dst_hbm.at[wid], dst_v)
    plsc.subcore_barrier()

    def body(j, carry):
        pltpu.sync_copy(ones_v, deg_sh.at[dst_v.at[j]], add=True)
        return carry

    lax.fori_loop(0, KCH, body, 0)
    plsc.subcore_barrier()
    pltpu.sync_copy(deg_sh.at[pl.ds(sid * RPT, RPT)],
                    out_hbm.at[pl.ds(cid * N_PAD + sid * RPT, RPT)])


_deg_call = functools.partial(
    pl.kernel,
    out_type=jax.ShapeDtypeStruct((2 * N_PAD,), jnp.float32),
    mesh=_sc_mesh,
    scratch_types=[
        pltpu.VMEM((KCH, CHUNK), jnp.int32),
        pltpu.VMEM((CHUNK,), jnp.float32),
        pltpu.VMEM_SHARED((N_PAD,), jnp.float32),
    ],
)(_deg_body)


NBUF = 2


def _scat_body(g_hbm, src_hbm, dst_hbm, zrows_hbm, out_hbm,
               src_v, dst_v, r0, r1, acc_sh, s0, s1):
    cid = lax.axis_index("c")
    sid = lax.axis_index("s")
    wid = sid * 2 + cid
    rows = (r0, r1)
    sems = (s0, s1)
    pltpu.sync_copy(zrows_hbm, acc_sh.at[pl.ds(sid * RPT, RPT)])
    plsc.subcore_barrier()

    for p in range(KCH // HKCH):
        pltpu.sync_copy(src_hbm.at[wid, pl.ds(p * HKCH, HKCH)], src_v)
        pltpu.sync_copy(dst_hbm.at[wid, pl.ds(p * HKCH, HKCH)], dst_v)
        for b in range(NBUF):
            pltpu.async_copy(g_hbm.at[src_v.at[b]], rows[b], sems[b])

        def body(jj, carry):
            for b in range(NBUF):
                j = jj * NBUF + b
                pltpu.make_async_copy(
                    g_hbm.at[src_v.at[j]], rows[b], sems[b]).wait()
                pltpu.sync_copy(rows[b], acc_sh.at[dst_v.at[j]], add=True)
                pltpu.async_copy(g_hbm.at[src_v.at[j + NBUF]], rows[b], sems[b])
            return carry

        lax.fori_loop(0, HKCH // NBUF - 1, body, 0)
        jl = HKCH - NBUF
        for b in range(NBUF):
            pltpu.make_async_copy(
                g_hbm.at[src_v.at[jl + b]], rows[b], sems[b]).wait()
            pltpu.sync_copy(rows[b], acc_sh.at[dst_v.at[jl + b]], add=True)
    plsc.subcore_barrier()
    pltpu.sync_copy(acc_sh.at[pl.ds(sid * RPT, RPT)],
                    out_hbm.at[pl.ds(cid * N_PAD + sid * RPT, RPT)])


_scat_call = functools.partial(
    pl.kernel,
    out_type=jax.ShapeDtypeStruct((2 * N_PAD, D), jnp.float32),
    mesh=_sc_mesh,
    scratch_types=[
        pltpu.VMEM((HKCH, CHUNK), jnp.int32),
        pltpu.VMEM((HKCH, CHUNK), jnp.int32),
        pltpu.VMEM((CHUNK, D), jnp.float32),
        pltpu.VMEM((CHUNK, D), jnp.float32),
        pltpu.VMEM_SHARED((N_PAD, D), jnp.float32),
        pltpu.SemaphoreType.DMA,
        pltpu.SemaphoreType.DMA,
    ],
)(_scat_body)


def _tc_a_body(x_ref, w1_ref, dega_ref, degb_ref, g_ref, dis_ref):
    deg = dega_ref[...] + degb_ref[...] + 1.0
    dis = lax.rsqrt(deg)
    h = jnp.dot(x_ref[...], w1_ref[...], preferred_element_type=jnp.float32)
    g_ref[...] = h * dis
    dis_ref[...] = dis


_tc_a = pl.pallas_call(
    _tc_a_body,
    out_shape=(jax.ShapeDtypeStruct((N_PAD, D), jnp.float32),
               jax.ShapeDtypeStruct((N_PAD, 1), jnp.float32)),
)


def _tc_c_body(acca_ref, accb_ref, g1_ref, dis_ref, b1_ref, w2_ref, g2_ref):
    dis = dis_ref[...]
    h1 = (acca_ref[...] + accb_ref[...] + g1_ref[...]) * dis + b1_ref[...]
    h1 = jnp.maximum(h1, 0.0)
    g2_ref[...] = jnp.dot(h1, w2_ref[...], preferred_element_type=jnp.float32) * dis


_tc_c = pl.pallas_call(
    _tc_c_body,
    out_shape=jax.ShapeDtypeStruct((N_PAD, D), jnp.float32),
)


def _tc_e_body(acca_ref, accb_ref, g2_ref, dis_ref, b2_ref,
               wa_ref, ba_ref, wf_ref, bf_ref, out_ref):
    dis = dis_ref[...]
    h2 = (acca_ref[...] + accb_ref[...] + g2_ref[...]) * dis + b2_ref[...]
    h2 = jnp.maximum(h2, 0.0)
    logits = jnp.dot(h2, wa_ref[...], preferred_element_type=jnp.float32) + ba_ref[...]
    row = lax.broadcasted_iota(jnp.int32, (N_PAD, 1), 0)
    logits = jnp.where(row < N_NODES, logits, -1e30)
    m = jnp.max(logits)
    w = jnp.exp(logits - m)
    s = jnp.sum(w)
    pooled = jnp.sum(w * h2, axis=0, keepdims=True) / s
    out_ref[...] = jnp.dot(pooled, wf_ref[...],
                           preferred_element_type=jnp.float32) + bf_ref[...]


_tc_e = pl.pallas_call(
    _tc_e_body,
    out_shape=jax.ShapeDtypeStruct((1, LABEL_DIM), jnp.float32),
)


def _pad_edges(v):
    v2 = v.reshape(NW, EPW)
    padc = jnp.broadcast_to(
        (N_NODES + jnp.arange(NW, dtype=jnp.int32))[:, None], (NW, PAD_PW))
    return jnp.concatenate([v2, padc], axis=1).reshape(NW, KCH, CHUNK)


def kernel(x, edge_index, W1, b1, W2, b2, Wa, ba, Wf, bf):
    src_p = _pad_edges(edge_index[0].astype(jnp.int32))
    dst_p = _pad_edges(edge_index[1].astype(jnp.int32))
    ones = jnp.ones((CHUNK,), jnp.float32)
    zvec = jnp.zeros((RPT,), jnp.float32)
    zrows = jnp.zeros((RPT, D), jnp.float32)

    deg2 = _deg_call(dst_p, ones, zvec)
    dega = deg2[:N_PAD].reshape(N_PAD, 1)
    degb = deg2[N_PAD:].reshape(N_PAD, 1)

    xp = jnp.pad(x, ((0, N_PAD - N_NODES), (0, 0)))
    g1, dis = _tc_a(xp, W1, dega, degb)

    acc1 = _scat_call(g1, src_p, dst_p, zrows)
    g2 = _tc_c(acc1[:N_PAD], acc1[N_PAD:], g1, dis, b1.reshape(1, D), W2)

    acc2 = _scat_call(g2, src_p, dst_p, zrows)
    out = _tc_e(acc2[:N_PAD], acc2[N_PAD:], g2, dis, b2.reshape(1, D),
                Wa, ba.reshape(1, 1), Wf, bf.reshape(1, LABEL_DIM))
    return out.reshape(LABEL_DIM)


# speedup vs baseline: 27.3503x; 1.0170x over previous
import functools

import jax
import jax.numpy as jnp
from jax import lax
from jax.experimental import pallas as pl
from jax.experimental.pallas import tpu as pltpu
from jax.experimental.pallas import tpu_sc as plsc

N_NODES = 10000
N_EDGES = 320000
D = 128
LABEL_DIM = 64

NW = 32
CHUNK = 128
KCH = 80
HKCH = 40
EPW = N_EDGES // NW
PAD_PW = KCH * CHUNK - EPW
N_PAD = 10240
RPT = N_PAD // 16

_sc_mesh = plsc.VectorSubcoreMesh(core_axis_name="c", subcore_axis_name="s")


def _deg_body(dst_hbm, ones_hbm, zvec_hbm, out_hbm, dst_v, ones_v, deg_sh):
    cid = lax.axis_index("c")
    sid = lax.axis_index("s")
    wid = sid * 2 + cid
    pltpu.sync_copy(zvec_hbm, deg_sh.at[pl.ds(sid * RPT, RPT)])
    pltpu.sync_copy(ones_hbm, ones_v)
    pltpu.sync_copy(dst_hbm.at[wid], dst_v)
    plsc.subcore_barrier()

    def body(j, carry):
        pltpu.sync_copy(ones_v, deg_sh.at[dst_v.at[j]], add=True)
        return carry

    lax.fori_loop(0, KCH, body, 0)
    plsc.subcore_barrier()
    pltpu.sync_copy(deg_sh.at[pl.ds(sid * RPT, RPT)],
                    out_hbm.at[pl.ds(cid * N_PAD + sid * RPT, RPT)])


_deg_call = functools.partial(
    pl.kernel,
    out_type=jax.ShapeDtypeStruct((2 * N_PAD,), jnp.float32),
    mesh=_sc_mesh,
    scratch_types=[
        pltpu.VMEM((KCH, CHUNK), jnp.int32),
        pltpu.VMEM((CHUNK,), jnp.float32),
        pltpu.VMEM_SHARED((N_PAD,), jnp.float32),
    ],
)(_deg_body)


NBUF = 2
PH = 16
NPHASE = KCH // PH


def _scat_body(g_hbm, src_hbm, dst_hbm, zrows_hbm, out_hbm,
               srcA, dstA, srcB, dstB, r0, r1, acc_sh, s0, s1, ss, sd):
    cid = lax.axis_index("c")
    sid = lax.axis_index("s")
    wid = sid * 2 + cid
    rows = (r0, r1)
    sems = (s0, s1)
    bufs = ((srcA, dstA), (srcB, dstB))

    pltpu.async_copy(src_hbm.at[wid, pl.ds(0, PH)], srcA, ss)
    pltpu.async_copy(dst_hbm.at[wid, pl.ds(0, PH)], dstA, sd)
    pltpu.sync_copy(zrows_hbm, acc_sh.at[pl.ds(sid * RPT, RPT)])
    plsc.subcore_barrier()
    pltpu.make_async_copy(src_hbm.at[wid, pl.ds(0, PH)], srcA, ss).wait()
    pltpu.make_async_copy(dst_hbm.at[wid, pl.ds(0, PH)], dstA, sd).wait()
    for b in range(NBUF):
        pltpu.async_copy(g_hbm.at[srcA.at[b]], rows[b], sems[b])

    for p in range(NPHASE):
        src_v, dst_v = bufs[p % 2]
        src_n, dst_n = bufs[(p + 1) % 2]
        if p + 1 < NPHASE:
            pltpu.async_copy(
                src_hbm.at[wid, pl.ds((p + 1) * PH, PH)], src_n, ss)
            pltpu.async_copy(
                dst_hbm.at[wid, pl.ds((p + 1) * PH, PH)], dst_n, sd)

        def body(jj, carry):
            for b in range(NBUF):
                j = jj * NBUF + b
                pltpu.make_async_copy(
                    g_hbm.at[src_v.at[j]], rows[b], sems[b]).wait()
                pltpu.sync_copy(rows[b], acc_sh.at[dst_v.at[j]], add=True)
                pltpu.async_copy(g_hbm.at[src_v.at[j + NBUF]], rows[b], sems[b])
            return carry

        lax.fori_loop(0, PH // NBUF - 1, body, 0)
        jl = PH - NBUF
        if p + 1 < NPHASE:
            pltpu.make_async_copy(
                src_hbm.at[wid, pl.ds((p + 1) * PH, PH)], src_n, ss).wait()
            pltpu.make_async_copy(
                dst_hbm.at[wid, pl.ds((p + 1) * PH, PH)], dst_n, sd).wait()
            for b in range(NBUF):
                pltpu.make_async_copy(
                    g_hbm.at[src_v.at[jl + b]], rows[b], sems[b]).wait()
                pltpu.sync_copy(rows[b], acc_sh.at[dst_v.at[jl + b]], add=True)
                pltpu.async_copy(g_hbm.at[src_n.at[b]], rows[b], sems[b])
        else:
            for b in range(NBUF):
                pltpu.make_async_copy(
                    g_hbm.at[src_v.at[jl + b]], rows[b], sems[b]).wait()
                pltpu.sync_copy(rows[b], acc_sh.at[dst_v.at[jl + b]], add=True)
    plsc.subcore_barrier()
    pltpu.sync_copy(acc_sh.at[pl.ds(sid * RPT, RPT)],
                    out_hbm.at[pl.ds(cid * N_PAD + sid * RPT, RPT)])


_scat_call = functools.partial(
    pl.kernel,
    out_type=jax.ShapeDtypeStruct((2 * N_PAD, D), jnp.float32),
    mesh=_sc_mesh,
    scratch_types=[
        pltpu.VMEM((PH, CHUNK), jnp.int32),
        pltpu.VMEM((PH, CHUNK), jnp.int32),
        pltpu.VMEM((PH, CHUNK), jnp.int32),
        pltpu.VMEM((PH, CHUNK), jnp.int32),
        pltpu.VMEM((CHUNK, D), jnp.float32),
        pltpu.VMEM((CHUNK, D), jnp.float32),
        pltpu.VMEM_SHARED((N_PAD, D), jnp.float32),
        pltpu.SemaphoreType.DMA,
        pltpu.SemaphoreType.DMA,
        pltpu.SemaphoreType.DMA,
        pltpu.SemaphoreType.DMA,
    ],
)(_scat_body)


def _tc_a_body(x_ref, w1_ref, dega_ref, degb_ref, g_ref, dis_ref):
    deg = dega_ref[...] + degb_ref[...] + 1.0
    dis = lax.rsqrt(deg)
    h = jnp.dot(x_ref[...], w1_ref[...], preferred_element_type=jnp.float32)
    g_ref[...] = h * dis
    dis_ref[...] = dis


_tc_a = pl.pallas_call(
    _tc_a_body,
    out_shape=(jax.ShapeDtypeStruct((N_PAD, D), jnp.float32),
               jax.ShapeDtypeStruct((N_PAD, 1), jnp.float32)),
)


def _tc_c_body(acca_ref, accb_ref, g1_ref, dis_ref, b1_ref, w2_ref, g2_ref):
    dis = dis_ref[...]
    h1 = (acca_ref[...] + accb_ref[...] + g1_ref[...]) * dis + b1_ref[...]
    h1 = jnp.maximum(h1, 0.0)
    g2_ref[...] = jnp.dot(h1, w2_ref[...], preferred_element_type=jnp.float32) * dis


_tc_c = pl.pallas_call(
    _tc_c_body,
    out_shape=jax.ShapeDtypeStruct((N_PAD, D), jnp.float32),
)


def _tc_e_body(acca_ref, accb_ref, g2_ref, dis_ref, b2_ref,
               wa_ref, ba_ref, wf_ref, bf_ref, out_ref):
    dis = dis_ref[...]
    h2 = (acca_ref[...] + accb_ref[...] + g2_ref[...]) * dis + b2_ref[...]
    h2 = jnp.maximum(h2, 0.0)
    logits = jnp.dot(h2, wa_ref[...], preferred_element_type=jnp.float32) + ba_ref[...]
    row = lax.broadcasted_iota(jnp.int32, (N_PAD, 1), 0)
    logits = jnp.where(row < N_NODES, logits, -1e30)
    m = jnp.max(logits)
    w = jnp.exp(logits - m)
    s = jnp.sum(w)
    pooled = jnp.sum(w * h2, axis=0, keepdims=True) / s
    out_ref[...] = jnp.dot(pooled, wf_ref[...],
                           preferred_element_type=jnp.float32) + bf_ref[...]


_tc_e = pl.pallas_call(
    _tc_e_body,
    out_shape=jax.ShapeDtypeStruct((1, LABEL_DIM), jnp.float32),
)


def _pad_edges(v):
    v2 = v.reshape(NW, EPW)
    padc = jnp.broadcast_to(
        (N_NODES + jnp.arange(NW, dtype=jnp.int32))[:, None], (NW, PAD_PW))
    return jnp.concatenate([v2, padc], axis=1).reshape(NW, KCH, CHUNK)


def kernel(x, edge_index, W1, b1, W2, b2, Wa, ba, Wf, bf):
    src_p = _pad_edges(edge_index[0].astype(jnp.int32))
    dst_p = _pad_edges(edge_index[1].astype(jnp.int32))
    ones = jnp.ones((CHUNK,), jnp.float32)
    zvec = jnp.zeros((RPT,), jnp.float32)
    zrows = jnp.zeros((RPT, D), jnp.float32)

    deg2 = _deg_call(dst_p, ones, zvec)
    dega = deg2[:N_PAD].reshape(N_PAD, 1)
    degb = deg2[N_PAD:].reshape(N_PAD, 1)

    xp = jnp.pad(x, ((0, N_PAD - N_NODES), (0, 0)))
    g1, dis = _tc_a(xp, W1, dega, degb)

    acc1 = _scat_call(g1, src_p, dst_p, zrows)
    g2 = _tc_c(acc1[:N_PAD], acc1[N_PAD:], g1, dis, b1.reshape(1, D), W2)

    acc2 = _scat_call(g2, src_p, dst_p, zrows)
    out = _tc_e(acc2[:N_PAD], acc2[N_PAD:], g2, dis, b2.reshape(1, D),
                Wa, ba.reshape(1, 1), Wf, bf.reshape(1, LABEL_DIM))
    return out.reshape(LABEL_DIM)
